# baseline (speedup 1.0000x reference)
import functools

import jax
import jax.numpy as jnp
from jax import lax
from jax.experimental import pallas as pl
from jax.experimental.pallas import tpu as pltpu
from jax.experimental.pallas import tpu_sc as plsc

E = 320000
D = 128
NC, NS = 2, 16
NW = NC * NS
CHUNK = E // NW
BLK = 80
NBLK = CHUNK // BLK
GPB = BLK // 16
N0, N1, N2 = 5, 6, 2
NCOMB = N0 * N1 * N2


def _sc_body(edge_hbm, t0_hbm, t1_hbm, t2_hbm, out_hbm, c_hbm,
             ebuf, tb0, tb1, tb2, cflat, cidx, rows,
             sem_g0, sem_g1, sem_s0, sem_s1):
    cid = lax.axis_index("c")
    sid = lax.axis_index("s")
    wid = sid * NC + cid
    ebase = wid * CHUNK

    pltpu.sync_copy(edge_hbm.at[pl.ds(ebase * 3, CHUNK * 3)], ebuf)

    @pl.when(sid == 0)
    def _():
        pltpu.sync_copy(t0_hbm, tb0)
        pltpu.sync_copy(t1_hbm, tb1)
        pltpu.sync_copy(t2_hbm, tb2)

        def build_row(c, carry):
            i0 = c // (N1 * N2)
            r = c - i0 * (N1 * N2)
            i1 = r // N2
            i2 = r - i1 * N2
            for j in range(D // 16):
                s = pl.ds(j * 16, 16)
                cflat[c, s] = tb0[i0, s] + tb1[i1, s] + tb2[i2, s]
            return carry
        lax.fori_loop(0, NCOMB, build_row, 0)
        pltpu.sync_copy(cflat, c_hbm)

    plsc.subcore_barrier()

    lanes = lax.iota(jnp.int32, 16)
    sem_g = (sem_g0, sem_g1)
    sem_s = (sem_s0, sem_s1)

    def compute_cidx(b, half):
        for g in range(GPB):
            posv = (b * BLK + g * 16) * 3 + lanes * 3
            a0 = plsc.load_gather(ebuf, [posv])
            a1 = plsc.load_gather(ebuf, [posv + 1])
            a2 = plsc.load_gather(ebuf, [posv + 2])
            cidx[half, pl.ds(g * 16, 16)] = a0 * (N1 * N2) + a1 * N2 + a2

    def gather_copy(b, half):
        return pltpu.make_async_copy(
            c_hbm.at[cidx.at[half]], rows.at[half], sem_g[half])

    def scatter_copy(b, half):
        return pltpu.make_async_copy(
            rows.at[half], out_hbm.at[pl.ds(ebase + b * BLK, BLK), :],
            sem_s[half])

    def do_block(b, half, p):
        @pl.when(p >= 1)
        def _():
            scatter_copy(b, half).wait()
        compute_cidx(b, half)
        gather_copy(b, half).start()
        gather_copy(b, half).wait()
        scatter_copy(b, half).start()

    def pair(p, carry):
        for half in (0, 1):
            do_block(p * 2 + half, half, p)
        return carry
    lax.fori_loop(0, NBLK // 2, pair, 0)

    do_block(NBLK - 1, 0, NBLK // 2)
    scatter_copy(NBLK - 2, 1).wait()
    scatter_copy(NBLK - 1, 0).wait()


@functools.partial(
    pl.kernel,
    out_type=(
        jax.ShapeDtypeStruct((E, D), jnp.float32),
        jax.ShapeDtypeStruct((NCOMB, D), jnp.float32),
    ),
    mesh=plsc.VectorSubcoreMesh(core_axis_name="c", subcore_axis_name="s"),
    compiler_params=pltpu.CompilerParams(needs_layout_passes=False),
    scratch_types=[
        pltpu.VMEM((CHUNK * 3,), jnp.int32),
        pltpu.VMEM((N0, D), jnp.float32),
        pltpu.VMEM((N1, D), jnp.float32),
        pltpu.VMEM((N2, D), jnp.float32),
        pltpu.VMEM((NCOMB, D), jnp.float32),
        pltpu.VMEM((2, BLK), jnp.int32),
        pltpu.VMEM((2, BLK, D), jnp.float32),
        pltpu.SemaphoreType.DMA,
        pltpu.SemaphoreType.DMA,
        pltpu.SemaphoreType.DMA,
        pltpu.SemaphoreType.DMA,
    ],
)
def _bond_encode_sc(edge_hbm, t0_hbm, t1_hbm, t2_hbm, out_hbm, c_hbm,
                    ebuf, tb0, tb1, tb2, cflat, cidx, rows,
                    sem_g0, sem_g1, sem_s0, sem_s1):
    _sc_body(edge_hbm, t0_hbm, t1_hbm, t2_hbm, out_hbm, c_hbm,
             ebuf, tb0, tb1, tb2, cflat, cidx, rows,
             sem_g0, sem_g1, sem_s0, sem_s1)


def kernel(edge_attr, table0, table1, table2):
    ea = edge_attr.astype(jnp.int32).reshape(-1)
    out, _ = _bond_encode_sc(ea, table0, table1, table2)
    return out


# speedup vs baseline: 3.8054x; 3.8054x over previous
import functools

import jax
import jax.numpy as jnp
from jax import lax
from jax.experimental import pallas as pl
from jax.experimental.pallas import tpu as pltpu
from jax.experimental.pallas import tpu_sc as plsc

E = 320000
D = 128
NC, NS = 2, 16
NW = NC * NS
CHUNK = E // NW
BLK = 80
NBLK = CHUNK // BLK
GPB = BLK // 16
N0, N1, N2 = 5, 6, 2
NCOMB = N0 * N1 * N2


def _sc_body(edge_hbm, t0_hbm, t1_hbm, t2_hbm, out_hbm,
             ebuf, tb0, tb1, tb2, cflat, cidx_v, obuf,
             sem_s0, sem_s1):
    wid = lax.axis_index("s") * NC + lax.axis_index("c")
    ebase = wid * CHUNK

    pltpu.sync_copy(edge_hbm.at[pl.ds(ebase * 3, CHUNK * 3)], ebuf)
    pltpu.sync_copy(t0_hbm, tb0)
    pltpu.sync_copy(t1_hbm, tb1)
    pltpu.sync_copy(t2_hbm, tb2)

    def build_row(c, carry):
        i0 = c // (N1 * N2)
        r = c - i0 * (N1 * N2)
        i1 = r // N2
        i2 = r - i1 * N2
        for j in range(D // 16):
            s = pl.ds(j * 16, 16)
            cflat[pl.ds(c * D + j * 16, 16)] = tb0[i0, s] + tb1[i1, s] + tb2[i2, s]
        return carry
    lax.fori_loop(0, NCOMB, build_row, 0)

    lanes = lax.iota(jnp.int32, 16)
    sem_s = (sem_s0, sem_s1)

    def compute_cidx(b, half):
        for g in range(GPB):
            posv = (b * BLK + g * 16) * 3 + lanes * 3
            a0 = plsc.load_gather(ebuf, [posv])
            a1 = plsc.load_gather(ebuf, [posv + 1])
            a2 = plsc.load_gather(ebuf, [posv + 2])
            cidx_v[half, pl.ds(g * 16, 16)] = a0 * (N1 * N2) + a1 * N2 + a2

    def fill_block(half):
        obase = half * (BLK * D)

        def gbody(g, carry):
            cvec = cidx_v[half, pl.ds(g * 16, 16)] * D
            rbase = obase + g * (16 * D)
            for lane in range(16):
                cbase = cvec[lane]
                for j in range(D // 16):
                    obuf[pl.ds(rbase + lane * D + j * 16, 16)] = \
                        cflat[pl.ds(cbase + j * 16, 16)]
            return carry
        lax.fori_loop(0, GPB, gbody, 0)

    def scatter_copy(b, half):
        return pltpu.make_async_copy(
            obuf.at[pl.ds(half * (BLK * D), BLK * D)],
            out_hbm.at[pl.ds((ebase + b * BLK) * D, BLK * D)],
            sem_s[half])

    def do_block(b, half, p):
        @pl.when(p >= 1)
        def _():
            scatter_copy(b, half).wait()
        compute_cidx(b, half)
        fill_block(half)
        scatter_copy(b, half).start()

    def pair(p, carry):
        for half in (0, 1):
            do_block(p * 2 + half, half, p)
        return carry
    lax.fori_loop(0, NBLK // 2, pair, 0)

    do_block(NBLK - 1, 0, NBLK // 2)
    scatter_copy(NBLK - 2, 1).wait()
    scatter_copy(NBLK - 1, 0).wait()


@functools.partial(
    pl.kernel,
    out_type=jax.ShapeDtypeStruct((E * D,), jnp.float32),
    mesh=plsc.VectorSubcoreMesh(core_axis_name="c", subcore_axis_name="s"),
    compiler_params=pltpu.CompilerParams(needs_layout_passes=False),
    scratch_types=[
        pltpu.VMEM((CHUNK * 3,), jnp.int32),
        pltpu.VMEM((N0, D), jnp.float32),
        pltpu.VMEM((N1, D), jnp.float32),
        pltpu.VMEM((N2, D), jnp.float32),
        pltpu.VMEM((NCOMB * D,), jnp.float32),
        pltpu.VMEM((2, BLK), jnp.int32),
        pltpu.VMEM((2 * BLK * D,), jnp.float32),
        pltpu.SemaphoreType.DMA,
        pltpu.SemaphoreType.DMA,
    ],
)
def _bond_encode_sc(edge_hbm, t0_hbm, t1_hbm, t2_hbm, out_hbm,
                    ebuf, tb0, tb1, tb2, cflat, cidx_v, obuf,
                    sem_s0, sem_s1):
    _sc_body(edge_hbm, t0_hbm, t1_hbm, t2_hbm, out_hbm,
             ebuf, tb0, tb1, tb2, cflat, cidx_v, obuf,
             sem_s0, sem_s1)


def kernel(edge_attr, table0, table1, table2):
    ea = edge_attr.astype(jnp.int32).reshape(-1)
    out_flat = _bond_encode_sc(ea, table0, table1, table2)
    return out_flat.reshape(E, D)
